# baseline (speedup 1.0000x reference)
import functools

import jax
import jax.numpy as jnp
from jax import lax
from jax.experimental import pallas as pl
from jax.experimental.pallas import tpu as pltpu
from jax.experimental.pallas import tpu_sc as plsc

_LANES = 16
_NUM_WORKERS = 32
_UNROLL = 8
_NSTAGES = 4


@functools.lru_cache(maxsize=None)
def _build(n: int, num_types: int):
    grain = _NSTAGES * _LANES * _UNROLL
    chunk = ((n + _NUM_WORKERS - 1) // _NUM_WORKERS + grain - 1) // grain * grain
    sub = chunk // _NSTAGES
    assert (n - chunk) % 8 == 0 and n >= chunk

    mesh = plsc.VectorSubcoreMesh(core_axis_name="c", subcore_axis_name="s")

    scratch = (
        [pltpu.VMEM((sub,), jnp.float32) for _ in range(_NSTAGES)]
        + [pltpu.VMEM((sub,), jnp.int32) for _ in range(_NSTAGES)]
        + [pltpu.VMEM((sub,), jnp.float32) for _ in range(_NSTAGES)]
        + [
            pltpu.VMEM((num_types,), jnp.float32),
            pltpu.VMEM((num_types,), jnp.float32),
        ]
        + [pltpu.SemaphoreType.DMA for _ in range(_NSTAGES)]
        + [pltpu.SemaphoreType.DMA]
    )

    @functools.partial(
        pl.kernel,
        mesh=mesh,
        compiler_params=pltpu.CompilerParams(needs_layout_passes=False),
        out_type=jax.ShapeDtypeStruct((n,), jnp.float32),
        scratch_types=scratch,
    )
    def scale_shift(x_hbm, t_hbm, s_hbm, b_hbm, out_hbm, *refs):
        xv = refs[:_NSTAGES]
        tv = refs[_NSTAGES : 2 * _NSTAGES]
        ov = refs[2 * _NSTAGES : 3 * _NSTAGES]
        sv, bv = refs[3 * _NSTAGES], refs[3 * _NSTAGES + 1]
        sems = refs[3 * _NSTAGES + 2 : 3 * _NSTAGES + 2 + _NSTAGES]
        semo = refs[3 * _NSTAGES + 2 + _NSTAGES]

        wid = lax.axis_index("s") * 2 + lax.axis_index("c")
        base = jnp.minimum(wid * chunk, n - chunk)

        cs = pltpu.async_copy(s_hbm, sv, sems[0])
        cb = pltpu.async_copy(b_hbm, bv, sems[0])
        cin = []
        for k in range(_NSTAGES):
            off = base + k * sub
            cin.append(
                (
                    pltpu.async_copy(x_hbm.at[pl.ds(off, sub)], xv[k], sems[k]),
                    pltpu.async_copy(t_hbm.at[pl.ds(off, sub)], tv[k], sems[k]),
                )
            )
        cs.wait()
        cb.wait()

        cout = []
        for k in range(_NSTAGES):
            cx, ct = cin[k]
            cx.wait()
            ct.wait()
            xk, tk, ok = xv[k], tv[k], ov[k]

            @plsc.parallel_loop(0, sub, step=_LANES, unroll=_UNROLL)
            def _(i, xk=xk, tk=tk, ok=ok):
                sl = pl.ds(i, _LANES)
                tt = tk[sl]
                ss = plsc.load_gather(sv, [tt])
                bb = plsc.load_gather(bv, [tt])
                ok[sl] = bb + ss * xk[sl]

            cout.append(
                pltpu.async_copy(ok, out_hbm.at[pl.ds(base + k * sub, sub)], semo)
            )

        for c in cout:
            c.wait()

    return scale_shift


def kernel(in_field, types, scales, shifts):
    n = in_field.shape[0]
    num_types = scales.shape[0]
    fn = _build(n, num_types)
    types = types.reshape(n)
    if types.dtype != jnp.int32:
        types = types.astype(jnp.int32)
    out = fn(
        in_field.reshape(n),
        types,
        scales.reshape(num_types),
        shifts.reshape(num_types),
    )
    return out.reshape(n, 1)


# speedup vs baseline: 1.0006x; 1.0006x over previous
import functools

import jax
import jax.numpy as jnp
from jax import lax
from jax.experimental import pallas as pl
from jax.experimental.pallas import tpu as pltpu
from jax.experimental.pallas import tpu_sc as plsc

_LANES = 16
_NUM_WORKERS = 32
_UNROLL = 16


@functools.lru_cache(maxsize=None)
def _build(n: int, num_types: int):
    grain = 2 * _LANES * _UNROLL
    chunk = ((n + _NUM_WORKERS - 1) // _NUM_WORKERS + grain - 1) // grain * grain
    sub = chunk // 2
    assert (n - chunk) % 8 == 0 and n >= chunk

    mesh = plsc.VectorSubcoreMesh(core_axis_name="c", subcore_axis_name="s")

    @functools.partial(
        pl.kernel,
        mesh=mesh,
        compiler_params=pltpu.CompilerParams(needs_layout_passes=False),
        out_type=jax.ShapeDtypeStruct((n,), jnp.float32),
        scratch_types=[
            pltpu.VMEM((sub,), jnp.float32),
            pltpu.VMEM((sub,), jnp.float32),
            pltpu.VMEM((sub,), jnp.int32),
            pltpu.VMEM((sub,), jnp.int32),
            pltpu.VMEM((sub,), jnp.float32),
            pltpu.VMEM((sub,), jnp.float32),
            pltpu.VMEM((num_types,), jnp.float32),
            pltpu.VMEM((num_types,), jnp.float32),
            pltpu.SemaphoreType.DMA,
            pltpu.SemaphoreType.DMA,
            pltpu.SemaphoreType.DMA,
        ],
    )
    def scale_shift(
        x_hbm, t_hbm, s_hbm, b_hbm, out_hbm,
        xv0, xv1, tv0, tv1, ov0, ov1, sv, bv, sem0, sem1, semo,
    ):
        wid = lax.axis_index("s") * 2 + lax.axis_index("c")
        base = jnp.minimum(wid * chunk, n - chunk)
        cs = pltpu.async_copy(s_hbm, sv, sem0)
        cb = pltpu.async_copy(b_hbm, bv, sem0)
        cx0 = pltpu.async_copy(x_hbm.at[pl.ds(base, sub)], xv0, sem0)
        ct0 = pltpu.async_copy(t_hbm.at[pl.ds(base, sub)], tv0, sem0)
        cx1 = pltpu.async_copy(x_hbm.at[pl.ds(base + sub, sub)], xv1, sem1)
        ct1 = pltpu.async_copy(t_hbm.at[pl.ds(base + sub, sub)], tv1, sem1)
        cs.wait()
        cb.wait()
        cx0.wait()
        ct0.wait()

        @plsc.parallel_loop(0, sub, step=_LANES, unroll=_UNROLL)
        def _(i):
            sl = pl.ds(i, _LANES)
            tt = tv0[sl]
            ss = plsc.load_gather(sv, [tt])
            bb = plsc.load_gather(bv, [tt])
            ov0[sl] = bb + ss * xv0[sl]

        co0 = pltpu.async_copy(ov0, out_hbm.at[pl.ds(base, sub)], semo)
        cx1.wait()
        ct1.wait()

        @plsc.parallel_loop(0, sub, step=_LANES, unroll=_UNROLL)
        def _(i):
            sl = pl.ds(i, _LANES)
            tt = tv1[sl]
            ss = plsc.load_gather(sv, [tt])
            bb = plsc.load_gather(bv, [tt])
            ov1[sl] = bb + ss * xv1[sl]

        co1 = pltpu.async_copy(ov1, out_hbm.at[pl.ds(base + sub, sub)], semo)
        co0.wait()
        co1.wait()

    return scale_shift


def kernel(in_field, types, scales, shifts):
    n = in_field.shape[0]
    num_types = scales.shape[0]
    fn = _build(n, num_types)
    types = types.reshape(n)
    if types.dtype != jnp.int32:
        types = types.astype(jnp.int32)
    out = fn(
        in_field.reshape(n),
        types,
        scales.reshape(num_types),
        shifts.reshape(num_types),
    )
    return out.reshape(n, 1)


# speedup vs baseline: 1.0152x; 1.0146x over previous
import functools

import jax
import jax.numpy as jnp
from jax import lax
from jax.experimental import pallas as pl
from jax.experimental.pallas import tpu as pltpu
from jax.experimental.pallas import tpu_sc as plsc

_LANES = 16
_NUM_WORKERS = 32
_UNROLL = 4


@functools.lru_cache(maxsize=None)
def _build(n: int, num_types: int):
    grain = 2 * _LANES * _UNROLL
    chunk = ((n + _NUM_WORKERS - 1) // _NUM_WORKERS + grain - 1) // grain * grain
    sub = chunk // 2
    assert (n - chunk) % 8 == 0 and n >= chunk

    mesh = plsc.VectorSubcoreMesh(core_axis_name="c", subcore_axis_name="s")

    @functools.partial(
        pl.kernel,
        mesh=mesh,
        compiler_params=pltpu.CompilerParams(needs_layout_passes=False),
        out_type=jax.ShapeDtypeStruct((n,), jnp.float32),
        scratch_types=[
            pltpu.VMEM((sub,), jnp.float32),
            pltpu.VMEM((sub,), jnp.float32),
            pltpu.VMEM((sub,), jnp.int32),
            pltpu.VMEM((sub,), jnp.int32),
            pltpu.VMEM((sub,), jnp.float32),
            pltpu.VMEM((sub,), jnp.float32),
            pltpu.VMEM((num_types,), jnp.float32),
            pltpu.VMEM((num_types,), jnp.float32),
            pltpu.SemaphoreType.DMA,
            pltpu.SemaphoreType.DMA,
            pltpu.SemaphoreType.DMA,
        ],
    )
    def scale_shift(
        x_hbm, t_hbm, s_hbm, b_hbm, out_hbm,
        xv0, xv1, tv0, tv1, ov0, ov1, sv, bv, sem0, sem1, semo,
    ):
        wid = lax.axis_index("s") * 2 + lax.axis_index("c")
        base = jnp.minimum(wid * chunk, n - chunk)
        cs = pltpu.async_copy(s_hbm, sv, sem0)
        cb = pltpu.async_copy(b_hbm, bv, sem0)
        cx0 = pltpu.async_copy(x_hbm.at[pl.ds(base, sub)], xv0, sem0)
        ct0 = pltpu.async_copy(t_hbm.at[pl.ds(base, sub)], tv0, sem0)
        cx1 = pltpu.async_copy(x_hbm.at[pl.ds(base + sub, sub)], xv1, sem1)
        ct1 = pltpu.async_copy(t_hbm.at[pl.ds(base + sub, sub)], tv1, sem1)
        cs.wait()
        cb.wait()
        cx0.wait()
        ct0.wait()

        @plsc.parallel_loop(0, sub, step=_LANES, unroll=_UNROLL)
        def _(i):
            sl = pl.ds(i, _LANES)
            tt = tv0[sl]
            ss = plsc.load_gather(sv, [tt])
            bb = plsc.load_gather(bv, [tt])
            ov0[sl] = bb + ss * xv0[sl]

        co0 = pltpu.async_copy(ov0, out_hbm.at[pl.ds(base, sub)], semo)
        cx1.wait()
        ct1.wait()

        @plsc.parallel_loop(0, sub, step=_LANES, unroll=_UNROLL)
        def _(i):
            sl = pl.ds(i, _LANES)
            tt = tv1[sl]
            ss = plsc.load_gather(sv, [tt])
            bb = plsc.load_gather(bv, [tt])
            ov1[sl] = bb + ss * xv1[sl]

        co1 = pltpu.async_copy(ov1, out_hbm.at[pl.ds(base + sub, sub)], semo)
        co0.wait()
        co1.wait()

    return scale_shift


def kernel(in_field, types, scales, shifts):
    n = in_field.shape[0]
    num_types = scales.shape[0]
    fn = _build(n, num_types)
    types = types.reshape(n)
    if types.dtype != jnp.int32:
        types = types.astype(jnp.int32)
    out = fn(
        in_field.reshape(n),
        types,
        scales.reshape(num_types),
        shifts.reshape(num_types),
    )
    return out.reshape(n, 1)
